# baseline (speedup 1.0000x reference)
import jax
import jax.numpy as jnp
from jax.experimental import pallas as pl
from jax.experimental.pallas import tpu as pltpu


_TILE_T = 512


def _add_pe_kernel(x_ref, emb_ref, out_ref):
    out_ref[...] = x_ref[...] + emb_ref[...][None, :, :]


def kernel(x, emb):
    b, t, d = x.shape
    grid = (t // _TILE_T,)
    return pl.pallas_call(
        _add_pe_kernel,
        grid=grid,
        in_specs=[
            pl.BlockSpec((b, _TILE_T, d), lambda i: (0, i, 0)),
            pl.BlockSpec((_TILE_T, d), lambda i: (i, 0)),
        ],
        out_specs=pl.BlockSpec((b, _TILE_T, d), lambda i: (0, i, 0)),
        out_shape=jax.ShapeDtypeStruct((b, t, d), x.dtype),
        compiler_params=pltpu.CompilerParams(
            dimension_semantics=("parallel",),
        ),
    )(x, emb[:t])
